# baseline (speedup 1.0000x reference)
import functools

import jax
import jax.numpy as jnp
from jax.experimental import pallas as pl

NUM_EXPERTS = 64
K = 8
HIGH_NEGATIVE = -100000.0
BLK_T = 1024
CHUNK = 256


def _gate_chunk(x, w, bias, noise):
    logits = jax.lax.dot_general(
        x, w, (((1,), (1,)), ((), ())), preferred_element_type=jnp.float32
    )
    logits = logits + bias + noise

    rows = logits.shape[0]
    v = logits
    c = jnp.zeros((rows, 1), jnp.int32)
    v8 = jnp.zeros((rows, 1), jnp.float32)
    v9 = jnp.zeros((rows, 1), jnp.float32)
    for _ in range(K + 1):
        m = jnp.max(v, axis=1, keepdims=True)
        eqm = v == m
        q = jnp.sum(eqm.astype(jnp.int32), axis=1, keepdims=True)
        c_after = c + q
        v8 = jnp.where((c < K) & (c_after >= K), m, v8)
        v9 = jnp.where((c < K + 1) & (c_after >= K + 1), m, v9)
        v = jnp.where(eqm, -jnp.inf, v)
        c = c_after

    quant = v9 + 0.125 * (v8 - v9)
    masked = jnp.where(logits > quant, logits, HIGH_NEGATIVE)
    rm = jnp.max(masked, axis=1, keepdims=True)
    p = jnp.exp(masked - rm)
    return p / jnp.sum(p, axis=1, keepdims=True)


def _gate_block(x_ref, w_ref, b_ref, noise_ref, o_ref):
    w = w_ref[...]
    bias = b_ref[...]
    for j in range(BLK_T // CHUNK):
        sl = pl.ds(j * CHUNK, CHUNK)
        o_ref[sl, :] = _gate_chunk(x_ref[sl, :], w, bias, noise_ref[sl, :])


@functools.partial(jax.jit, static_argnames=())
def kernel(input, W, b, noise):
    tokens, d_model = input.shape
    b2 = b.reshape(1, NUM_EXPERTS)
    grid = (tokens // BLK_T,)
    return pl.pallas_call(
        _gate_block,
        grid=grid,
        in_specs=[
            pl.BlockSpec((BLK_T, d_model), lambda i: (i, 0)),
            pl.BlockSpec((NUM_EXPERTS, d_model), lambda i: (0, 0)),
            pl.BlockSpec((1, NUM_EXPERTS), lambda i: (0, 0)),
            pl.BlockSpec((BLK_T, NUM_EXPERTS), lambda i: (i, 0)),
        ],
        out_specs=pl.BlockSpec((BLK_T, NUM_EXPERTS), lambda i: (i, 0)),
        out_shape=jax.ShapeDtypeStruct((tokens, NUM_EXPERTS), jnp.float32),
    )(input, W, b2, noise)


# speedup vs baseline: 1.2318x; 1.2318x over previous
import functools

import jax
import jax.numpy as jnp
from jax.experimental import pallas as pl

NUM_EXPERTS = 64
K = 8
HIGH_NEGATIVE = -100000.0
BLK_T = 512
CHUNK = 256


def _gate_chunk(x, w, bias, noise):
    logits = jax.lax.dot_general(
        x, w, (((1,), (1,)), ((), ())), preferred_element_type=jnp.float32
    )
    logits = logits + bias + noise

    rows = logits.shape[0]
    lt = logits.T
    v = lt
    c = jnp.zeros((1, rows), jnp.float32)
    v1 = jnp.zeros((1, rows), jnp.float32)
    v8 = jnp.zeros((1, rows), jnp.float32)
    v9 = jnp.zeros((1, rows), jnp.float32)
    for i in range(K + 1):
        m = jnp.max(v, axis=0, keepdims=True)
        if i == 0:
            v1 = m
        eqm = v == m
        q = jnp.sum(jnp.where(eqm, 1.0, 0.0), axis=0, keepdims=True)
        c_after = c + q
        v8 = jnp.where((c < float(K)) & (c_after >= float(K)), m, v8)
        v9 = jnp.where((c < float(K + 1)) & (c_after >= float(K + 1)), m, v9)
        v = jnp.where(eqm, -jnp.inf, v)
        c = c_after

    quant_t = v9 + 0.125 * (v8 - v9)
    rm_t = jnp.where(v1 > quant_t, v1, HIGH_NEGATIVE)
    quant = quant_t.reshape(rows, 1)
    rm = rm_t.reshape(rows, 1)
    masked = jnp.where(logits > quant, logits, HIGH_NEGATIVE)
    p = jnp.exp(masked - rm)
    return p / jnp.sum(p, axis=1, keepdims=True)


def _gate_block(x_ref, w_ref, b_ref, noise_ref, o_ref):
    w = w_ref[...]
    bias = b_ref[...]
    for j in range(BLK_T // CHUNK):
        sl = pl.ds(j * CHUNK, CHUNK)
        o_ref[sl, :] = _gate_chunk(x_ref[sl, :], w, bias, noise_ref[sl, :])


@functools.partial(jax.jit, static_argnames=())
def kernel(input, W, b, noise):
    tokens, d_model = input.shape
    b2 = b.reshape(1, NUM_EXPERTS)
    grid = (tokens // BLK_T,)
    return pl.pallas_call(
        _gate_block,
        grid=grid,
        in_specs=[
            pl.BlockSpec((BLK_T, d_model), lambda i: (i, 0)),
            pl.BlockSpec((NUM_EXPERTS, d_model), lambda i: (0, 0)),
            pl.BlockSpec((1, NUM_EXPERTS), lambda i: (0, 0)),
            pl.BlockSpec((BLK_T, NUM_EXPERTS), lambda i: (i, 0)),
        ],
        out_specs=pl.BlockSpec((BLK_T, NUM_EXPERTS), lambda i: (i, 0)),
        out_shape=jax.ShapeDtypeStruct((tokens, NUM_EXPERTS), jnp.float32),
    )(input, W, b2, noise)
